# baseline (speedup 1.0000x reference)
import functools

import jax
import jax.numpy as jnp
from jax import lax
from jax.experimental import pallas as pl
from jax.experimental.pallas import tpu as pltpu
from jax.experimental.pallas import tpu_sc as plsc

BATCH = 4096
HIST = 200
PROD_D = 16
ACT_D = 8
OUT_D = PROD_D + ACT_D
TOTAL = BATCH * HIST

_info = plsc.get_sparse_core_info()
NC = _info.num_cores
NS = _info.num_subcores
NW = NC * NS
PER_W = TOTAL // NW
CHUNK = 3200
NCHUNK = PER_W // CHUNK

_mesh = plsc.VectorSubcoreMesh(core_axis_name="c", subcore_axis_name="s")


@functools.partial(
    pl.kernel,
    mesh=_mesh,
    out_type=jax.ShapeDtypeStruct((TOTAL, OUT_D), jnp.float32),
    scratch_types=[
        pltpu.VMEM((CHUNK,), jnp.int32),
        pltpu.VMEM((CHUNK,), jnp.int32),
        pltpu.VMEM((CHUNK, PROD_D), jnp.float32),
        pltpu.VMEM((CHUNK, ACT_D), jnp.float32),
        pltpu.SemaphoreType.DMA,
        pltpu.SemaphoreType.DMA,
    ],
    compiler_params=pltpu.CompilerParams(use_tc_tiling_on_sc=False),
)
def _gather_concat(xp_hbm, xa_hbm, ptab_hbm, atab_hbm, out_hbm,
                   idxp_v, idxa_v, prod_v, act_v, semp, sema):
    wid = lax.axis_index("s") * NC + lax.axis_index("c")
    wbase = wid * PER_W
    for ci in range(NCHUNK):
        base = wbase + ci * CHUNK
        pltpu.sync_copy(xp_hbm.at[pl.ds(base, CHUNK)], idxp_v)
        pltpu.sync_copy(xa_hbm.at[pl.ds(base, CHUNK)], idxa_v)
        cp = pltpu.async_copy(ptab_hbm.at[idxp_v], prod_v, semp)
        ca = pltpu.async_copy(atab_hbm.at[idxa_v], act_v, sema)
        cp.wait()
        ca.wait()
        pltpu.sync_copy(prod_v, out_hbm.at[pl.ds(base, CHUNK), pl.ds(0, PROD_D)])
        pltpu.sync_copy(act_v, out_hbm.at[pl.ds(base, CHUNK), pl.ds(PROD_D, ACT_D)])


def kernel(x_prod, x_act, product_table, action_table):
    xp = jnp.reshape(x_prod, (TOTAL,)).astype(jnp.int32)
    xa = jnp.reshape(x_act, (TOTAL,)).astype(jnp.int32)
    out = _gather_concat(xp, xa, product_table, action_table)
    return jnp.reshape(out, (BATCH, HIST, OUT_D))


def _smoke():
    key = jax.random.key(0)
    k1, k2, k3, k4 = jax.random.split(key, 4)
    xp = jax.random.randint(k1, (BATCH, HIST), 0, 1000000, dtype=jnp.int32)
    xa = jax.random.randint(k2, (BATCH, HIST), 0, 20, dtype=jnp.int32)
    pt = jax.random.normal(k3, (1000000, PROD_D), dtype=jnp.float32)
    at = jax.random.normal(k4, (20, ACT_D), dtype=jnp.float32)
    out = kernel(xp, xa, pt, at)
    print(out.shape, out.dtype)


if __name__ == "__main__":
    _smoke()


# speedup vs baseline: 1.0012x; 1.0012x over previous
import functools

import jax
import jax.numpy as jnp
from jax import lax
from jax.experimental import pallas as pl
from jax.experimental.pallas import tpu as pltpu
from jax.experimental.pallas import tpu_sc as plsc

BATCH = 4096
HIST = 200
PROD_D = 16
ACT_D = 8
OUT_D = PROD_D + ACT_D
TOTAL = BATCH * HIST

_info = plsc.get_sparse_core_info()
NC = _info.num_cores
NS = _info.num_subcores
NW = NC * NS
PER_W = TOTAL // NW
CHUNK = 1600
NCHUNK = PER_W // CHUNK

_mesh = plsc.VectorSubcoreMesh(core_axis_name="c", subcore_axis_name="s")


@functools.partial(
    pl.kernel,
    mesh=_mesh,
    out_type=jax.ShapeDtypeStruct((TOTAL, OUT_D), jnp.float32),
    scratch_types=[
        pltpu.VMEM((CHUNK,), jnp.int32),
        pltpu.VMEM((CHUNK,), jnp.int32),
        pltpu.VMEM((CHUNK, PROD_D), jnp.float32),
        pltpu.VMEM((CHUNK, ACT_D), jnp.float32),
        pltpu.SemaphoreType.DMA,
        pltpu.SemaphoreType.DMA,
    ],
    compiler_params=pltpu.CompilerParams(use_tc_tiling_on_sc=False),
)
def _gather_concat(xp_hbm, xa_hbm, ptab_hbm, atab_hbm, out_hbm,
                   idxp_v, idxa_v, prod_v, act_v, semp, sema):
    wid = lax.axis_index("s") * NC + lax.axis_index("c")
    wbase = wid * PER_W
    for ci in range(NCHUNK):
        base = wbase + ci * CHUNK
        pltpu.sync_copy(xp_hbm.at[pl.ds(base, CHUNK)], idxp_v)
        pltpu.sync_copy(xa_hbm.at[pl.ds(base, CHUNK)], idxa_v)
        cp = pltpu.async_copy(ptab_hbm.at[idxp_v], prod_v, semp)
        ca = pltpu.async_copy(atab_hbm.at[idxa_v], act_v, sema)
        cp.wait()
        ca.wait()
        pltpu.sync_copy(prod_v, out_hbm.at[pl.ds(base, CHUNK), pl.ds(0, PROD_D)])
        pltpu.sync_copy(act_v, out_hbm.at[pl.ds(base, CHUNK), pl.ds(PROD_D, ACT_D)])


def kernel(x_prod, x_act, product_table, action_table):
    xp = jnp.reshape(x_prod, (TOTAL,)).astype(jnp.int32)
    xa = jnp.reshape(x_act, (TOTAL,)).astype(jnp.int32)
    out = _gather_concat(xp, xa, product_table, action_table)
    return jnp.reshape(out, (BATCH, HIST, OUT_D))


def _smoke():
    key = jax.random.key(0)
    k1, k2, k3, k4 = jax.random.split(key, 4)
    xp = jax.random.randint(k1, (BATCH, HIST), 0, 1000000, dtype=jnp.int32)
    xa = jax.random.randint(k2, (BATCH, HIST), 0, 20, dtype=jnp.int32)
    pt = jax.random.normal(k3, (1000000, PROD_D), dtype=jnp.float32)
    at = jax.random.normal(k4, (20, ACT_D), dtype=jnp.float32)
    out = kernel(xp, xa, pt, at)
    print(out.shape, out.dtype)


if __name__ == "__main__":
    _smoke()


# speedup vs baseline: 3.9122x; 3.9075x over previous
import functools

import jax
import jax.numpy as jnp
from jax import lax
from jax.experimental import pallas as pl
from jax.experimental.pallas import tpu as pltpu
from jax.experimental.pallas import tpu_sc as plsc

BATCH = 4096
HIST = 200
PROD_D = 16
ACT_D = 8
OUT_D = PROD_D + ACT_D
TOTAL = BATCH * HIST

_info = plsc.get_sparse_core_info()
NC = _info.num_cores
NS = _info.num_subcores
NW = NC * NS
ROWS_PER_W = BATCH // NW
RCHUNK = 16
CHUNK = RCHUNK * HIST
NCHUNK = ROWS_PER_W // RCHUNK

_OFFS = tuple(range(0, HIST - 16, 16)) + (HIST - 16,)

_mesh = plsc.VectorSubcoreMesh(core_axis_name="c", subcore_axis_name="s")


@functools.partial(
    pl.kernel,
    mesh=_mesh,
    out_type=jax.ShapeDtypeStruct((TOTAL, OUT_D), jnp.float32),
    scratch_types=[
        pltpu.VMEM((RCHUNK, HIST), jnp.int32),
        pltpu.VMEM((RCHUNK, HIST), jnp.int32),
        pltpu.VMEM((CHUNK, PROD_D), jnp.float32),
        pltpu.VMEM((CHUNK, ACT_D), jnp.float32),
        pltpu.VMEM((ACT_D, 20), jnp.float32),
        pltpu.SemaphoreType.DMA,
    ],
    compiler_params=pltpu.CompilerParams(use_tc_tiling_on_sc=False,
                                         needs_layout_passes=False),
)
def _gather_concat(xp_hbm, xa_hbm, ptab_hbm, atabt_hbm, out_hbm,
                   idxp_v, idxa_v, prod_v, act_v, atabt_v, semp):
    wid = lax.axis_index("s") * NC + lax.axis_index("c")
    row0 = wid * ROWS_PER_W
    pltpu.sync_copy(atabt_hbm, atabt_v)
    lane = lax.iota(jnp.int32, 16)
    for ci in range(NCHUNK):
        r = row0 + ci * RCHUNK
        base = r * HIST
        pltpu.sync_copy(xp_hbm.at[pl.ds(r, RCHUNK), :], idxp_v)
        pltpu.sync_copy(xa_hbm.at[pl.ds(r, RCHUNK), :], idxa_v)
        cps = [
            pltpu.async_copy(ptab_hbm.at[idxp_v.at[rr, :]],
                             prod_v.at[pl.ds(rr * HIST, HIST), :], semp)
            for rr in range(RCHUNK)
        ]

        def act_body(rr, _):
            rbase = rr * HIST
            for off in _OFFS:
                aidx = idxa_v[rr, pl.ds(off, 16)]
                rows = rbase + off + lane
                for c in range(ACT_D):
                    cvec = jnp.full((16,), c, jnp.int32)
                    vals = plsc.load_gather(atabt_v, [cvec, aidx])
                    plsc.store_scatter(act_v, [rows, cvec], vals)
            return 0

        lax.fori_loop(0, RCHUNK, act_body, 0)

        for cp in cps:
            cp.wait()
        pltpu.sync_copy(prod_v,
                        out_hbm.at[pl.ds(base, CHUNK), pl.ds(0, PROD_D)])
        pltpu.sync_copy(act_v,
                        out_hbm.at[pl.ds(base, CHUNK), pl.ds(PROD_D, ACT_D)])


def kernel(x_prod, x_act, product_table, action_table):
    atabt = jnp.transpose(action_table)
    out = _gather_concat(x_prod, x_act, product_table, atabt)
    return jnp.reshape(out, (BATCH, HIST, OUT_D))


def _smoke():
    key = jax.random.key(0)
    k1, k2, k3, k4 = jax.random.split(key, 4)
    xp = jax.random.randint(k1, (BATCH, HIST), 0, 1000000, dtype=jnp.int32)
    xa = jax.random.randint(k2, (BATCH, HIST), 0, 20, dtype=jnp.int32)
    pt = jax.random.normal(k3, (1000000, PROD_D), dtype=jnp.float32)
    at = jax.random.normal(k4, (20, ACT_D), dtype=jnp.float32)
    out = kernel(xp, xa, pt, at)
    print(out.shape, out.dtype)


if __name__ == "__main__":
    _smoke()


# speedup vs baseline: 3.9297x; 1.0045x over previous
import functools

import jax
import jax.numpy as jnp
from jax import lax
from jax.experimental import pallas as pl
from jax.experimental.pallas import tpu as pltpu
from jax.experimental.pallas import tpu_sc as plsc

BATCH = 4096
HIST = 200
PROD_D = 16
ACT_D = 8
OUT_D = PROD_D + ACT_D
TOTAL = BATCH * HIST
LANES = 128
IROWS = TOTAL // LANES

_info = plsc.get_sparse_core_info()
NC = _info.num_cores
NS = _info.num_subcores
NW = NC * NS
ROWS_PER_W = IROWS // NW
RCHUNK = 25
CHUNK = RCHUNK * LANES
NCHUNK = ROWS_PER_W // RCHUNK

_mesh = plsc.VectorSubcoreMesh(core_axis_name="c", subcore_axis_name="s")


@functools.partial(
    pl.kernel,
    mesh=_mesh,
    out_type=jax.ShapeDtypeStruct((TOTAL, OUT_D), jnp.float32),
    scratch_types=[
        pltpu.VMEM((RCHUNK, LANES), jnp.int32),
        pltpu.VMEM((RCHUNK, LANES), jnp.int32),
        pltpu.VMEM((CHUNK, PROD_D), jnp.float32),
        pltpu.VMEM((CHUNK, ACT_D), jnp.float32),
        pltpu.VMEM((ACT_D, 20), jnp.float32),
        pltpu.SemaphoreType.DMA,
    ],
    compiler_params=pltpu.CompilerParams(use_tc_tiling_on_sc=False,
                                         needs_layout_passes=False),
)
def _gather_concat(xp_hbm, xa_hbm, ptab_hbm, atabt_hbm, out_hbm,
                   idxp_v, idxa_v, prod_v, act_v, atabt_v, semp):
    wid = lax.axis_index("s") * NC + lax.axis_index("c")
    row0 = wid * ROWS_PER_W
    pltpu.sync_copy(atabt_hbm, atabt_v)
    lane = lax.iota(jnp.int32, 16)
    for ci in range(NCHUNK):
        r = row0 + ci * RCHUNK
        base = r * LANES
        pltpu.sync_copy(xp_hbm.at[pl.ds(r, RCHUNK), :], idxp_v)
        pltpu.sync_copy(xa_hbm.at[pl.ds(r, RCHUNK), :], idxa_v)
        cps = [
            pltpu.async_copy(ptab_hbm.at[idxp_v.at[rr, :]],
                             prod_v.at[pl.ds(rr * LANES, LANES), :], semp)
            for rr in range(RCHUNK)
        ]

        def act_body(rr, _):
            rbase = rr * LANES
            for off in range(0, LANES, 16):
                aidx = idxa_v[rr, pl.ds(off, 16)]
                rows = rbase + off + lane
                for c in range(ACT_D):
                    cvec = jnp.full((16,), c, jnp.int32)
                    vals = plsc.load_gather(atabt_v, [cvec, aidx])
                    plsc.store_scatter(act_v, [rows, cvec], vals)
            return 0

        lax.fori_loop(0, RCHUNK, act_body, 0)

        for cp in cps:
            cp.wait()
        pltpu.sync_copy(prod_v,
                        out_hbm.at[pl.ds(base, CHUNK), pl.ds(0, PROD_D)])
        pltpu.sync_copy(act_v,
                        out_hbm.at[pl.ds(base, CHUNK), pl.ds(PROD_D, ACT_D)])


def kernel(x_prod, x_act, product_table, action_table):
    xp = jnp.reshape(x_prod.astype(jnp.int32), (IROWS, LANES))
    xa = jnp.reshape(x_act.astype(jnp.int32), (IROWS, LANES))
    atabt = jnp.transpose(action_table)
    out = _gather_concat(xp, xa, product_table, atabt)
    return jnp.reshape(out, (BATCH, HIST, OUT_D))


def _smoke():
    key = jax.random.key(0)
    k1, k2, k3, k4 = jax.random.split(key, 4)
    xp = jax.random.randint(k1, (BATCH, HIST), 0, 1000000, dtype=jnp.int32)
    xa = jax.random.randint(k2, (BATCH, HIST), 0, 20, dtype=jnp.int32)
    pt = jax.random.normal(k3, (1000000, PROD_D), dtype=jnp.float32)
    at = jax.random.normal(k4, (20, ACT_D), dtype=jnp.float32)
    out = kernel(xp, xa, pt, at)
    print(out.shape, out.dtype)


if __name__ == "__main__":
    _smoke()
